# Initial kernel scaffold; baseline (speedup 1.0000x reference)
import jax
import jax.numpy as jnp
from jax.experimental import pallas as pl


def kernel(coords, sum_log_probs, tour, tgt_len, tgt_tour, attn_matrix):
    raise NotImplementedError("write your pallas kernel here")


# speedup vs baseline: 2.1327x; 2.1327x over previous
import jax
import jax.numpy as jnp
from jax.experimental import pallas as pl
from jax.experimental.pallas import tpu as pltpu

_N = 50
_BB = 128
_TINY = 1.1754943508222875e-38


def _body(attn_ref, tour_ref, gtt_ref, xs_ref, ys_ref, tlen_ref, out_ref,
          *, batch_total):
    i = pl.program_id(0)
    pltpu.prng_seed(i)

    a = attn_ref[:]
    la = jnp.log(a)

    ent = jnp.sum(la * a, axis=-1)
    w = (jax.lax.broadcasted_iota(jnp.int32, (_BB, _N), 1) + 1).astype(jnp.float32) \
        * jnp.float32(2.0 / (_N * (_N + 1)))
    h_part = -jnp.sum(ent * w)

    bits = pltpu.prng_random_bits((_BB, _N, _N))
    ubits = jax.lax.bitcast_convert_type(bits, jnp.uint32)
    fbits = (ubits >> jnp.uint32(9)) | jnp.uint32(0x3F800000)
    f = jax.lax.bitcast_convert_type(fbits, jnp.float32) - 1.0
    u = jnp.maximum(jnp.float32(_TINY), f * jnp.float32(1.0 - _TINY) + jnp.float32(_TINY))
    g = -jnp.log(-jnp.log(u))
    z = la + g
    zmax = jnp.max(z, axis=-1, keepdims=True)
    jidx = jax.lax.broadcasted_iota(jnp.int32, (_BB, _N, _N), 2)
    s = jnp.min(jnp.where(z >= zmax, jidx, _N), axis=-1)
    onehot = jidx == s[:, :, None]
    logp = jnp.sum(jnp.where(onehot, la, 0.0), axis=-1)
    gt = gtt_ref[:, :_N] - 1
    match = s == gt

    tour = tour_ref[:]
    cmp = jidx == tour[:, :, None]
    tcx = jnp.sum(jnp.where(cmp, xs_ref[:][:, None, :], 0.0), axis=-1)
    tcy = jnp.sum(jnp.where(cmp, ys_ref[:][:, None, :], 0.0), axis=-1)
    nx = jnp.concatenate([tcx[:, 1:], tcx[:, :1]], axis=1)
    ny = jnp.concatenate([tcy[:, 1:], tcy[:, :1]], axis=1)
    dx = nx - tcx
    dy = ny - tcy
    tour_len = jnp.sum(jnp.sqrt(dx * dx + dy * dy + 1e-12), axis=1)
    tlen = tlen_ref[:, 0]
    mask = tour_len < tlen
    diffv = tour_len - tlen

    r = jnp.where(mask[:, None], -1.0,
                  jnp.where(match, -1.0, 0.0))
    last = jnp.where(mask, -10.0 + diffv, r[:, -1])
    ncol = jax.lax.broadcasted_iota(jnp.int32, (_BB, _N), 1)
    r = jnp.where(ncol == _N - 1, last[:, None], r)

    tri = (jax.lax.broadcasted_iota(jnp.int32, (_N, _N), 0)
           <= jax.lax.broadcasted_iota(jnp.int32, (_N, _N), 1)
           ).astype(jnp.float32)
    c = jnp.dot(r, tri, preferred_element_type=jnp.float32)
    loss_part = jnp.sum(c * logp)

    total = (0.6 / (batch_total * _N)) * loss_part + (0.4 / batch_total) * h_part

    @pl.when(i == 0)
    def _init():
        out_ref[:, :] = jnp.zeros((1, 1), jnp.float32)
    out_ref[:, :] += total[None, None]


def kernel(coords, sum_log_probs, tour, tgt_len, tgt_tour, attn_matrix):
    del sum_log_probs
    b = attn_matrix.shape[0]
    assert b % _BB == 0
    xs = coords[:, :, 0]
    ys = coords[:, :, 1]
    tlen2 = tgt_len[:, None]

    import functools
    out = pl.pallas_call(
        functools.partial(_body, batch_total=b),
        grid=(b // _BB,),
        in_specs=[
            pl.BlockSpec((_BB, _N, _N), lambda i: (i, 0, 0)),
            pl.BlockSpec((_BB, _N), lambda i: (i, 0)),
            pl.BlockSpec((_BB, _N + 1), lambda i: (i, 0)),
            pl.BlockSpec((_BB, _N), lambda i: (i, 0)),
            pl.BlockSpec((_BB, _N), lambda i: (i, 0)),
            pl.BlockSpec((_BB, 1), lambda i: (i, 0)),
        ],
        out_specs=pl.BlockSpec((1, 1), lambda i: (0, 0)),
        out_shape=jax.ShapeDtypeStruct((1, 1), jnp.float32),
    )(attn_matrix, tour, tgt_tour, xs, ys, tlen2)
    return out.reshape(())


# speedup vs baseline: 2.7722x; 1.2999x over previous
import functools

import jax
import jax.numpy as jnp
from jax.experimental import pallas as pl
from jax.experimental.pallas import tpu as pltpu

_N = 50
_C = _N * _N
_BF = 256
_TINY = 1.1754943508222875e-38


def _body(attn_ref, tour_ref, gtt_ref, xs_ref, ys_ref, tlen_ref, out_ref,
          *, batch_total):
    i = pl.program_id(0)
    pltpu.prng_seed(i)

    a = attn_ref[:]
    la = jnp.log(a)

    lane = jax.lax.broadcasted_iota(jnp.int32, (1, _C), 1)
    seg = lane // _N
    jloc = lane - seg * _N

    wvec = (seg + 1).astype(jnp.float32) * jnp.float32(2.0 / (_N * (_N + 1)))
    h_part = -jnp.sum(la * a * wvec)

    bits = pltpu.prng_random_bits((_BF, _C))
    ubits = jax.lax.bitcast_convert_type(bits, jnp.uint32)
    fbits = (ubits >> jnp.uint32(9)) | jnp.uint32(0x3F800000)
    f = jax.lax.bitcast_convert_type(fbits, jnp.float32) - 1.0
    u = jnp.maximum(jnp.float32(_TINY),
                    f * jnp.float32(1.0 - _TINY) + jnp.float32(_TINY))
    g = -jnp.log(-jnp.log(u))
    z = la + g

    mz, ml = z, la
    for s in (1, 2, 4, 8, 16, 18):
        ok = (jloc + s) < _N
        rz = jnp.roll(mz, -s, axis=1)
        rl = jnp.roll(ml, -s, axis=1)
        take = ok & (rz > mz)
        mz = jnp.where(take, rz, mz)
        ml = jnp.where(take, rl, ml)

    crow = jax.lax.broadcasted_iota(jnp.int32, (_C, _N), 0)
    ncol = jax.lax.broadcasted_iota(jnp.int32, (_C, _N), 1)
    smat = (crow // _N == ncol).astype(jnp.float32)

    start = (jloc == 0).astype(jnp.float32)
    picked = jnp.concatenate([mz * start, ml * start], axis=0)
    packed = jnp.dot(picked, smat, preferred_element_type=jnp.float32)
    zmaxn = packed[:_BF]
    lpn = packed[_BF:]

    gt = (gtt_ref[:, :_N] - 1).astype(jnp.float32)
    stm = (jax.lax.broadcasted_iota(jnp.int32, (_N, _C), 0)
           == jax.lax.broadcasted_iota(jnp.int32, (_N, _C), 1) // _N
           ).astype(jnp.float32)
    gtb = jnp.dot(gt, stm, preferred_element_type=jnp.float32)
    zg = jnp.where(gtb == jloc.astype(jnp.float32), z, 0.0)
    zgt = jnp.dot(zg, smat, preferred_element_type=jnp.float32)
    match = zgt == zmaxn

    tour = tour_ref[:]
    kidx = jax.lax.broadcasted_iota(jnp.int32, (_BF, _N, _N), 2)
    cmp = kidx == tour[:, :, None]
    tcx = jnp.sum(jnp.where(cmp, xs_ref[:][:, None, :], 0.0), axis=-1)
    tcy = jnp.sum(jnp.where(cmp, ys_ref[:][:, None, :], 0.0), axis=-1)
    nx = jnp.concatenate([tcx[:, 1:], tcx[:, :1]], axis=1)
    ny = jnp.concatenate([tcy[:, 1:], tcy[:, :1]], axis=1)
    dx = nx - tcx
    dy = ny - tcy
    tour_len = jnp.sum(jnp.sqrt(dx * dx + dy * dy + 1e-12), axis=1)
    tlen = tlen_ref[:, 0]
    mask = tour_len < tlen
    diffv = tour_len - tlen

    r = jnp.where(mask[:, None], -1.0,
                  jnp.where(match, -1.0, 0.0))
    last = jnp.where(mask, -10.0 + diffv, r[:, -1])
    col = jax.lax.broadcasted_iota(jnp.int32, (_BF, _N), 1)
    r = jnp.where(col == _N - 1, last[:, None], r)

    tri = (jax.lax.broadcasted_iota(jnp.int32, (_N, _N), 0)
           <= jax.lax.broadcasted_iota(jnp.int32, (_N, _N), 1)
           ).astype(jnp.float32)
    c = jnp.dot(r, tri, preferred_element_type=jnp.float32)
    loss_part = jnp.sum(c * lpn)

    total = (0.6 / (batch_total * _N)) * loss_part + (0.4 / batch_total) * h_part

    @pl.when(i == 0)
    def _init():
        out_ref[:, :] = jnp.zeros((1, 1), jnp.float32)
    out_ref[:, :] += total[None, None]


def kernel(coords, sum_log_probs, tour, tgt_len, tgt_tour, attn_matrix):
    del sum_log_probs
    b = attn_matrix.shape[0]
    assert b % _BF == 0
    attn_flat = attn_matrix.reshape(b, _C)
    xs = coords[:, :, 0]
    ys = coords[:, :, 1]
    tlen2 = tgt_len[:, None]

    out = pl.pallas_call(
        functools.partial(_body, batch_total=b),
        grid=(b // _BF,),
        in_specs=[
            pl.BlockSpec((_BF, _C), lambda i: (i, 0)),
            pl.BlockSpec((_BF, _N), lambda i: (i, 0)),
            pl.BlockSpec((_BF, _N + 1), lambda i: (i, 0)),
            pl.BlockSpec((_BF, _N), lambda i: (i, 0)),
            pl.BlockSpec((_BF, _N), lambda i: (i, 0)),
            pl.BlockSpec((_BF, 1), lambda i: (i, 0)),
        ],
        out_specs=pl.BlockSpec((1, 1), lambda i: (0, 0)),
        out_shape=jax.ShapeDtypeStruct((1, 1), jnp.float32),
    )(attn_flat, tour, tgt_tour, xs, ys, tlen2)
    return out.reshape(())
